# baseline (speedup 1.0000x reference)
import functools

import jax
import jax.numpy as jnp
from jax import lax
from jax.experimental import pallas as pl
from jax.experimental.pallas import tpu as pltpu
from jax.experimental.pallas import tpu_sc as plsc

_N = 10000
_E = 320000
_D = 128
_EPS = 1e-5
_BNS = 1.0 / (1.0 + _EPS) ** 0.5

_NC = 2
_NS = 16
_NW = _NC * _NS
_CH = 128
_NB = 2
_NBI = 4
_NPAD = 10240
_RPT = _NPAD // _NS
_NCHK = 80
_NCHKT = _NCHK + _NBI
_EP = _NCHK * _CH
_EPAD = _EP * _NW

_BM = 256
_NG = _NPAD // _BM


def _sc_mesh():
    return plsc.VectorSubcoreMesh(
        core_axis_name="c", subcore_axis_name="s",
        num_cores=_NC, num_subcores=_NS)


@functools.partial(
    pl.kernel,
    out_type=jax.ShapeDtypeStruct((_NC, _NPAD), jnp.float32),
    mesh=_sc_mesh(),
    scratch_types=[
        pltpu.VMEM((_NCHK, _CH), jnp.int32),
        pltpu.VMEM((_CH,), jnp.float32),
        pltpu.VMEM((_RPT,), jnp.float32),
        pltpu.VMEM_SHARED((_NPAD,), jnp.float32),
    ],
)
def _deg(dstp_hbm, degp_hbm, dloc, ones_v, zb, dacc):
    c = lax.axis_index("c")
    s = lax.axis_index("s")
    wid = s * _NC + c
    z16 = jnp.zeros((16,), jnp.float32)
    o16 = jnp.ones((16,), jnp.float32)

    def fz(i, _):
        zb[pl.ds(i * 16, 16)] = z16
        return 0
    lax.fori_loop(0, _RPT // 16, fz, 0)

    def fo(i, _):
        ones_v[pl.ds(i * 16, 16)] = o16
        return 0
    lax.fori_loop(0, _CH // 16, fo, 0)

    pltpu.sync_copy(dstp_hbm.at[wid], dloc)
    pltpu.sync_copy(zb, dacc.at[pl.ds(s * _RPT, _RPT)])
    plsc.subcore_barrier()

    def step(g, _):
        pltpu.sync_copy(ones_v, dacc.at[dloc.at[g]], add=True)
        return 0
    lax.fori_loop(0, _NCHK, step, 0)

    plsc.subcore_barrier()
    sl = pl.ds(s * _RPT, _RPT)
    pltpu.sync_copy(dacc.at[sl], degp_hbm.at[c, sl])


@functools.partial(
    pl.kernel,
    out_type=jax.ShapeDtypeStruct((_NC, _NPAD, _D), jnp.float32),
    mesh=_sc_mesh(),
    scratch_types=[
        pltpu.VMEM((_CH,), jnp.int32),
        pltpu.VMEM((_CH,), jnp.int32),
        pltpu.VMEM((_CH, _D), jnp.float32),
        pltpu.VMEM_SHARED((_NPAD, _D), jnp.float32),
    ],
)
def _agg(hp_hbm, srcp_hbm, dstp_hbm, out_hbm, sidx, didx, rows, acc):
    c = lax.axis_index("c")
    s = lax.axis_index("s")
    wid = s * _NC + c
    z16 = jnp.zeros((16,), jnp.float32)

    def fz(i, _):
        rows[i // (_D // 16), pl.ds((i % (_D // 16)) * 16, 16)] = z16
        return 0
    lax.fori_loop(0, _CH * _D // 16, fz, 0)
    for j in range(_RPT // _CH):
        pltpu.sync_copy(rows, acc.at[pl.ds(s * _RPT + j * _CH, _CH)])
    plsc.subcore_barrier()

    eb = wid * _EP

    def step(g, _):
        off = eb + g * _CH
        pltpu.sync_copy(srcp_hbm.at[pl.ds(off, _CH)], sidx)
        pltpu.sync_copy(dstp_hbm.at[pl.ds(off, _CH)], didx)
        pltpu.sync_copy(hp_hbm.at[sidx], rows)
        pltpu.sync_copy(rows, acc.at[didx], add=True)
        return 0
    lax.fori_loop(0, _NCHK, step, 0)

    plsc.subcore_barrier()
    for j in range(_RPT // _CH):
        sl = pl.ds(s * _RPT + j * _CH, _CH)
        pltpu.sync_copy(acc.at[sl], out_hbm.at[c, sl])


def _b0_body(x_ref, degp_ref, w_ref, b_ref, o_ref):
    i = pl.program_id(0)
    deg = degp_ref[0] + degp_ref[1] + 1.0
    dinv = lax.rsqrt(deg)
    h = jnp.dot(x_ref[...], w_ref[...],
                preferred_element_type=jnp.float32) + b_ref[...]
    rowid = i * _BM + lax.broadcasted_iota(jnp.int32, (_BM, 1), 0)
    o_ref[...] = jnp.where(rowid < _N, dinv[:, None] * h, 0.0)


_b0 = pl.pallas_call(
    _b0_body,
    grid=(_NG,),
    in_specs=[
        pl.BlockSpec((_BM, _D), lambda i: (i, 0)),
        pl.BlockSpec((_NC, _BM), lambda i: (0, i)),
        pl.BlockSpec((_D, _D), lambda i: (0, 0)),
        pl.BlockSpec((1, _D), lambda i: (0, 0)),
    ],
    out_specs=pl.BlockSpec((_BM, _D), lambda i: (i, 0)),
    out_shape=jax.ShapeDtypeStruct((_NPAD, _D), jnp.float32),
)


def _mid_body(p_ref, hp_ref, degp_ref, g_ref, be_ref, w_ref, b_ref, o_ref):
    i = pl.program_id(0)
    deg = degp_ref[0] + degp_ref[1] + 1.0
    dinv = lax.rsqrt(deg)
    agg = p_ref[0] + p_ref[1] + hp_ref[...]
    y = jnp.maximum(g_ref[...] * _BNS * (dinv[:, None] * agg) + be_ref[...],
                    0.0)
    h = jnp.dot(y, w_ref[...], preferred_element_type=jnp.float32) + b_ref[...]
    rowid = i * _BM + lax.broadcasted_iota(jnp.int32, (_BM, 1), 0)
    o_ref[...] = jnp.where(rowid < _N, dinv[:, None] * h, 0.0)


_mid = pl.pallas_call(
    _mid_body,
    grid=(_NG,),
    in_specs=[
        pl.BlockSpec((_NC, _BM, _D), lambda i: (0, i, 0)),
        pl.BlockSpec((_BM, _D), lambda i: (i, 0)),
        pl.BlockSpec((_NC, _BM), lambda i: (0, i)),
        pl.BlockSpec((1, _D), lambda i: (0, 0)),
        pl.BlockSpec((1, _D), lambda i: (0, 0)),
        pl.BlockSpec((_D, _D), lambda i: (0, 0)),
        pl.BlockSpec((1, _D), lambda i: (0, 0)),
    ],
    out_specs=pl.BlockSpec((_BM, _D), lambda i: (i, 0)),
    out_shape=jax.ShapeDtypeStruct((_NPAD, _D), jnp.float32),
)


def _fin_body(p_ref, hp_ref, degp_ref, g_ref, be_ref, fw1_ref, fb1_ref,
              fg_ref, fbe_ref, fw2_ref, fb2_ref, o_ref, acc_ref):
    i = pl.program_id(0)
    deg = degp_ref[0] + degp_ref[1] + 1.0
    dinv = lax.rsqrt(deg)
    agg = p_ref[0] + p_ref[1] + hp_ref[...]
    y = jnp.maximum(g_ref[...] * _BNS * (dinv[:, None] * agg) + be_ref[...],
                    0.0)
    rowid = i * _BM + lax.broadcasted_iota(jnp.int32, (_BM, 1), 0)
    y = jnp.where(rowid < _N, y, 0.0)
    part = jnp.sum(y, axis=0, keepdims=True)

    @pl.when(i == 0)
    def _():
        acc_ref[...] = jnp.zeros_like(acc_ref)

    acc_ref[...] += part

    @pl.when(i == _NG - 1)
    def _():
        pooled = acc_ref[...] * (1.0 / _N)
        z = jnp.dot(pooled, fw1_ref[...],
                    preferred_element_type=jnp.float32) + fb1_ref[...]
        z = jnp.maximum(fg_ref[...] * _BNS * z + fbe_ref[...], 0.0)
        o_ref[...] = jnp.sum(z * fw2_ref[...], axis=1,
                             keepdims=True) + fb2_ref[...]


_fin = pl.pallas_call(
    _fin_body,
    grid=(_NG,),
    in_specs=[
        pl.BlockSpec((_NC, _BM, _D), lambda i: (0, i, 0)),
        pl.BlockSpec((_BM, _D), lambda i: (i, 0)),
        pl.BlockSpec((_NC, _BM), lambda i: (0, i)),
        pl.BlockSpec((1, _D), lambda i: (0, 0)),
        pl.BlockSpec((1, _D), lambda i: (0, 0)),
        pl.BlockSpec((_D, _D // 2), lambda i: (0, 0)),
        pl.BlockSpec((1, _D // 2), lambda i: (0, 0)),
        pl.BlockSpec((1, _D // 2), lambda i: (0, 0)),
        pl.BlockSpec((1, _D // 2), lambda i: (0, 0)),
        pl.BlockSpec((1, _D // 2), lambda i: (0, 0)),
        pl.BlockSpec((1, 1), lambda i: (0, 0)),
    ],
    out_specs=pl.BlockSpec((1, 1), lambda i: (0, 0)),
    out_shape=jax.ShapeDtypeStruct((1, 1), jnp.float32),
    scratch_shapes=[pltpu.VMEM((1, _D), jnp.float32)],
)


def kernel(x, edge_index, W0, b0, g0, be0, W1, b1, g1, be1, W2, b2, g2, be2,
           fW1, fb1, fg, fbe, fW2, fb2):
    src = edge_index[0].astype(jnp.int32)
    dst = edge_index[1].astype(jnp.int32)
    pad = jnp.full((_EPAD - _E + _CH,), _N, jnp.int32)
    srcp = jnp.concatenate([src, pad])
    dstp = jnp.concatenate([dst, pad])
    dstp3 = dstp[:_EPAD].reshape(_NW, _NCHK, _CH)
    xp = jnp.pad(x, ((0, _NPAD - _N), (0, 0)))

    degp = _deg(dstp3)
    hp = _b0(xp, degp, W0, b0.reshape(1, -1))
    for (W, b, g, be) in ((W1, b1, g0, be0), (W2, b2, g1, be1)):
        P = _agg(hp, srcp, dstp)
        hp = _mid(P, hp, degp, g.reshape(1, -1), be.reshape(1, -1),
                  W, b.reshape(1, -1))
    P = _agg(hp, srcp, dstp)
    out = _fin(P, hp, degp, g2.reshape(1, -1), be2.reshape(1, -1),
               fW1, fb1.reshape(1, -1), fg.reshape(1, -1), fbe.reshape(1, -1),
               fW2.reshape(1, -1), fb2.reshape(1, 1))
    return out


# speedup vs baseline: 2.3575x; 2.3575x over previous
import functools

import jax
import jax.numpy as jnp
from jax import lax
from jax.experimental import pallas as pl
from jax.experimental.pallas import tpu as pltpu
from jax.experimental.pallas import tpu_sc as plsc

_N = 10000
_E = 320000
_D = 128
_EPS = 1e-5
_BNS = 1.0 / (1.0 + _EPS) ** 0.5

_NC = 2
_NS = 16
_NW = _NC * _NS
_CH = 128
_NB = 2
_NBI = 4
_NPAD = 10240
_RPT = _NPAD // _NS
_NCHK = 80
_NCHKT = _NCHK + _NBI
_EP = _NCHK * _CH
_EPAD = _EP * _NW

_BM = 256
_NG = _NPAD // _BM


def _sc_mesh():
    return plsc.VectorSubcoreMesh(
        core_axis_name="c", subcore_axis_name="s",
        num_cores=_NC, num_subcores=_NS)


@functools.partial(
    pl.kernel,
    out_type=jax.ShapeDtypeStruct((_NC, _NPAD), jnp.float32),
    mesh=_sc_mesh(),
    scratch_types=[
        pltpu.VMEM((_NCHK, _CH), jnp.int32),
        pltpu.VMEM((_CH,), jnp.float32),
        pltpu.VMEM((_RPT,), jnp.float32),
        pltpu.VMEM_SHARED((_NPAD,), jnp.float32),
    ],
)
def _deg(dstp_hbm, degp_hbm, dloc, ones_v, zb, dacc):
    c = lax.axis_index("c")
    s = lax.axis_index("s")
    wid = s * _NC + c
    z16 = jnp.zeros((16,), jnp.float32)
    o16 = jnp.ones((16,), jnp.float32)

    def fz(i, _):
        zb[pl.ds(i * 16, 16)] = z16
        return 0
    lax.fori_loop(0, _RPT // 16, fz, 0)

    def fo(i, _):
        ones_v[pl.ds(i * 16, 16)] = o16
        return 0
    lax.fori_loop(0, _CH // 16, fo, 0)

    pltpu.sync_copy(dstp_hbm.at[wid], dloc)
    pltpu.sync_copy(zb, dacc.at[pl.ds(s * _RPT, _RPT)])
    plsc.subcore_barrier()

    def step(g, _):
        pltpu.sync_copy(ones_v, dacc.at[dloc.at[g]], add=True)
        return 0
    lax.fori_loop(0, _NCHK, step, 0)

    plsc.subcore_barrier()
    sl = pl.ds(s * _RPT, _RPT)
    pltpu.sync_copy(dacc.at[sl], degp_hbm.at[c, sl])


@functools.partial(
    pl.kernel,
    out_type=jax.ShapeDtypeStruct((_NC, _NPAD, _D), jnp.float32),
    mesh=_sc_mesh(),
    scratch_types=[
        pltpu.VMEM((_CH,), jnp.int32),
        pltpu.VMEM((_CH,), jnp.int32),
        pltpu.VMEM((_CH, _D), jnp.float32),
        pltpu.VMEM_SHARED((_NPAD, _D), jnp.float32),
    ],
)
def _agg(hp_hbm, srcp_hbm, dstp_hbm, out_hbm, sidx, didx, rows, acc):
    c = lax.axis_index("c")
    s = lax.axis_index("s")
    wid = s * _NC + c
    z16 = jnp.zeros((16,), jnp.float32)

    def fz(i, _):
        rows[i // (_D // 16), pl.ds((i % (_D // 16)) * 16, 16)] = z16
        return 0
    lax.fori_loop(0, _CH * _D // 16, fz, 0)
    for j in range(_RPT // _CH):
        pltpu.sync_copy(rows, acc.at[pl.ds(s * _RPT + j * _CH, _CH)])
    plsc.subcore_barrier()

    eb = wid * _EP

    def step(g, _):
        off = eb + g * _CH
        pltpu.sync_copy(srcp_hbm.at[pl.ds(off, _CH)], sidx)
        pltpu.sync_copy(dstp_hbm.at[pl.ds(off, _CH)], didx)
        pltpu.sync_copy(hp_hbm.at[sidx], rows)
        pltpu.sync_copy(rows, acc.at[didx], add=True)
        return 0
    lax.fori_loop(0, _NCHK, step, 0)

    plsc.subcore_barrier()
    for j in range(_RPT // _CH):
        sl = pl.ds(s * _RPT + j * _CH, _CH)
        pltpu.sync_copy(acc.at[sl], out_hbm.at[c, sl])


def _b0_body(x_ref, degp_ref, w_ref, b_ref, o_ref):
    i = pl.program_id(0)
    deg = degp_ref[0] + degp_ref[1] + 1.0
    dinv = lax.rsqrt(deg)
    h = jnp.dot(x_ref[...], w_ref[...],
                preferred_element_type=jnp.float32) + b_ref[...]
    rowid = i * _BM + lax.broadcasted_iota(jnp.int32, (_BM, 1), 0)
    o_ref[...] = jnp.where(rowid < _N, dinv[:, None] * h, 0.0)


_b0 = pl.pallas_call(
    _b0_body,
    grid=(_NG,),
    in_specs=[
        pl.BlockSpec((_BM, _D), lambda i: (i, 0)),
        pl.BlockSpec((_NC, _BM), lambda i: (0, i)),
        pl.BlockSpec((_D, _D), lambda i: (0, 0)),
        pl.BlockSpec((1, _D), lambda i: (0, 0)),
    ],
    out_specs=pl.BlockSpec((_BM, _D), lambda i: (i, 0)),
    out_shape=jax.ShapeDtypeStruct((_NPAD, _D), jnp.float32),
)


def _mid_body(p_ref, hp_ref, degp_ref, g_ref, be_ref, w_ref, b_ref, o_ref):
    i = pl.program_id(0)
    deg = degp_ref[0] + degp_ref[1] + 1.0
    dinv = lax.rsqrt(deg)
    agg = p_ref[0] + p_ref[1] + hp_ref[...]
    y = jnp.maximum(g_ref[...] * _BNS * (dinv[:, None] * agg) + be_ref[...],
                    0.0)
    h = jnp.dot(y, w_ref[...], preferred_element_type=jnp.float32) + b_ref[...]
    rowid = i * _BM + lax.broadcasted_iota(jnp.int32, (_BM, 1), 0)
    o_ref[...] = jnp.where(rowid < _N, dinv[:, None] * h, 0.0)


_mid = pl.pallas_call(
    _mid_body,
    grid=(_NG,),
    in_specs=[
        pl.BlockSpec((_NC, _BM, _D), lambda i: (0, i, 0)),
        pl.BlockSpec((_BM, _D), lambda i: (i, 0)),
        pl.BlockSpec((_NC, _BM), lambda i: (0, i)),
        pl.BlockSpec((1, _D), lambda i: (0, 0)),
        pl.BlockSpec((1, _D), lambda i: (0, 0)),
        pl.BlockSpec((_D, _D), lambda i: (0, 0)),
        pl.BlockSpec((1, _D), lambda i: (0, 0)),
    ],
    out_specs=pl.BlockSpec((_BM, _D), lambda i: (i, 0)),
    out_shape=jax.ShapeDtypeStruct((_NPAD, _D), jnp.float32),
)


def _fin_body(p_ref, hp_ref, degp_ref, g_ref, be_ref, fw1_ref, fb1_ref,
              fg_ref, fbe_ref, fw2_ref, fb2_ref, o_ref, acc_ref):
    i = pl.program_id(0)
    deg = degp_ref[0] + degp_ref[1] + 1.0
    dinv = lax.rsqrt(deg)
    agg = p_ref[0] + p_ref[1] + hp_ref[...]
    y = jnp.maximum(g_ref[...] * _BNS * (dinv[:, None] * agg) + be_ref[...],
                    0.0)
    rowid = i * _BM + lax.broadcasted_iota(jnp.int32, (_BM, 1), 0)
    y = jnp.where(rowid < _N, y, 0.0)
    part = jnp.sum(y, axis=0, keepdims=True)

    @pl.when(i == 0)
    def _():
        acc_ref[...] = jnp.zeros_like(acc_ref)

    acc_ref[...] += part

    @pl.when(i == _NG - 1)
    def _():
        pooled = acc_ref[...] * (1.0 / _N)
        z = jnp.dot(pooled, fw1_ref[...],
                    preferred_element_type=jnp.float32) + fb1_ref[...]
        z = jnp.maximum(fg_ref[...] * _BNS * z + fbe_ref[...], 0.0)
        o_ref[...] = jnp.sum(z * fw2_ref[...], axis=1,
                             keepdims=True) + fb2_ref[...]


_fin = pl.pallas_call(
    _fin_body,
    grid=(_NG,),
    in_specs=[
        pl.BlockSpec((_NC, _BM, _D), lambda i: (0, i, 0)),
        pl.BlockSpec((_BM, _D), lambda i: (i, 0)),
        pl.BlockSpec((_NC, _BM), lambda i: (0, i)),
        pl.BlockSpec((1, _D), lambda i: (0, 0)),
        pl.BlockSpec((1, _D), lambda i: (0, 0)),
        pl.BlockSpec((_D, _D // 2), lambda i: (0, 0)),
        pl.BlockSpec((1, _D // 2), lambda i: (0, 0)),
        pl.BlockSpec((1, _D // 2), lambda i: (0, 0)),
        pl.BlockSpec((1, _D // 2), lambda i: (0, 0)),
        pl.BlockSpec((1, _D // 2), lambda i: (0, 0)),
        pl.BlockSpec((1, 1), lambda i: (0, 0)),
    ],
    out_specs=pl.BlockSpec((1, 1), lambda i: (0, 0)),
    out_shape=jax.ShapeDtypeStruct((1, 1), jnp.float32),
    scratch_shapes=[pltpu.VMEM((1, _D), jnp.float32)],
)


def kernel(x, edge_index, W0, b0, g0, be0, W1, b1, g1, be1, W2, b2, g2, be2,
           fW1, fb1, fg, fbe, fW2, fb2):
    src = edge_index[0].astype(jnp.int32)
    dst = edge_index[1].astype(jnp.int32)
    npad_e = _EPAD - _E + _CH
    pad = _N + (jnp.arange(npad_e, dtype=jnp.int32) % (_NPAD - _N))
    srcp = jnp.concatenate([src, pad])
    dstp = jnp.concatenate([dst, pad])
    dstp3 = dstp[:_EPAD].reshape(_NW, _NCHK, _CH)
    xp = jnp.pad(x, ((0, _NPAD - _N), (0, 0)))

    degp = _deg(dstp3)
    hp = _b0(xp, degp, W0, b0.reshape(1, -1))
    for (W, b, g, be) in ((W1, b1, g0, be0), (W2, b2, g1, be1)):
        P = _agg(hp, srcp, dstp)
        hp = _mid(P, hp, degp, g.reshape(1, -1), be.reshape(1, -1),
                  W, b.reshape(1, -1))
    P = _agg(hp, srcp, dstp)
    out = _fin(P, hp, degp, g2.reshape(1, -1), be2.reshape(1, -1),
               fW1, fb1.reshape(1, -1), fg.reshape(1, -1), fbe.reshape(1, -1),
               fW2.reshape(1, -1), fb2.reshape(1, 1))
    return out
